# baseline (speedup 1.0000x reference)
import functools
import jax
import jax.numpy as jnp
from jax import lax
from jax.experimental import pallas as pl
from jax.experimental.pallas import tpu as pltpu, tpu_sc as plsc

_SEQ, _HID = 8192, 1024
_NC, _NS = 2, 16
_NW = _NC * _NS
_ROWS_PER_W = _SEQ // _NW
_CHUNK = 32
_NCHUNK = _ROWS_PER_W // _CHUNK

_mesh = plsc.VectorSubcoreMesh(core_axis_name="c", subcore_axis_name="s")


@functools.partial(
    pl.kernel,
    mesh=_mesh,
    out_type=jax.ShapeDtypeStruct((_SEQ, _HID), jnp.float32),
    scratch_types=[
        pltpu.VMEM((_CHUNK, _HID), jnp.float32),
        pltpu.VMEM((_CHUNK, _HID), jnp.float32),
        pltpu.SemaphoreType.DMA,
        pltpu.SemaphoreType.DMA,
        pltpu.SemaphoreType.DMA,
        pltpu.SemaphoreType.DMA,
    ],
)
def _sc_copy(table_hbm, out_hbm, buf0, buf1, isem0, isem1, osem0, osem1):
    wid = lax.axis_index("s") * _NC + lax.axis_index("c")
    base = wid * _ROWS_PER_W
    bufs = (buf0, buf1)
    isems = (isem0, isem1)
    osems = (osem0, osem1)
    in_c = []
    out_c = []
    for j in range(_NCHUNK):
        b = j % 2
        src = table_hbm.at[pl.ds(base + j * _CHUNK, _CHUNK)]
        dst = out_hbm.at[pl.ds(base + j * _CHUNK, _CHUNK)]
        in_c.append(pltpu.make_async_copy(src, bufs[b], isems[b]))
        out_c.append(pltpu.make_async_copy(bufs[b], dst, osems[b]))
    in_c[0].start()
    for j in range(_NCHUNK):
        if j >= 1:
            out_c[j - 1].wait()
        if j + 1 < _NCHUNK:
            in_c[j + 1].start()
        in_c[j].wait()
        out_c[j].start()
    out_c[_NCHUNK - 1].wait()


def kernel(input_ids, table):
    out = _sc_copy(table)
    return out[None]
